# Initial kernel scaffold; baseline (speedup 1.0000x reference)
import jax
import jax.numpy as jnp
from jax.experimental import pallas as pl


def kernel(atom_types, elemental_energies):
    raise NotImplementedError("write your pallas kernel here")


# speedup vs baseline: 24.8787x; 24.8787x over previous
import functools

import jax
import jax.numpy as jnp
from jax import lax
from jax.experimental import pallas as pl
from jax.experimental.pallas import tpu as pltpu
from jax.experimental.pallas import tpu_sc as plsc

_N = 100000
_T = 95
_L = 16
_NC = 2
_NS = 16
_NW = _NC * _NS
_CHUNK = 3136
_ROWS = _CHUNK // _L

_mesh = plsc.VectorSubcoreMesh(core_axis_name="c", subcore_axis_name="s")


@functools.partial(
    pl.kernel,
    mesh=_mesh,
    out_type=jax.ShapeDtypeStruct((_N,), jnp.float32),
    compiler_params=pltpu.CompilerParams(needs_layout_passes=False),
    scratch_types=[
        pltpu.VMEM((_T,), jnp.float32),
        pltpu.VMEM((_CHUNK,), jnp.int32),
        pltpu.VMEM((_CHUNK,), jnp.float32),
    ],
)
def _atomref_sc(types_hbm, table_hbm, out_hbm, table_v, idx_v, out_v):
    wid = lax.axis_index("s") * _NC + lax.axis_index("c")
    base = lax.min(wid * _CHUNK, _N - _CHUNK)

    pltpu.sync_copy(table_hbm, table_v)
    pltpu.sync_copy(types_hbm.at[pl.ds(base, _CHUNK)], idx_v)

    def body(i, carry):
        idx = idx_v[pl.ds(i * _L, _L)]
        out_v[pl.ds(i * _L, _L)] = plsc.load_gather(table_v, [idx])
        return carry

    lax.fori_loop(0, _ROWS, body, 0)

    pltpu.sync_copy(out_v, out_hbm.at[pl.ds(base, _CHUNK)])


def kernel(atom_types, elemental_energies):
    return _atomref_sc(atom_types.astype(jnp.int32), elemental_energies)


# speedup vs baseline: 26.2128x; 1.0536x over previous
import functools

import jax
import jax.numpy as jnp
from jax import lax
from jax.experimental import pallas as pl
from jax.experimental.pallas import tpu as pltpu
from jax.experimental.pallas import tpu_sc as plsc

_N = 100000
_T = 95
_L = 16
_NC = 2
_NS = 16
_NW = _NC * _NS
_CHUNK = 3136
_ROWS = _CHUNK // _L

_mesh = plsc.VectorSubcoreMesh(core_axis_name="c", subcore_axis_name="s")


@functools.partial(
    pl.kernel,
    mesh=_mesh,
    out_type=jax.ShapeDtypeStruct((_N,), jnp.float32),
    compiler_params=pltpu.CompilerParams(needs_layout_passes=False),
    scratch_types=[
        pltpu.VMEM((_T,), jnp.float32),
        pltpu.VMEM((_CHUNK,), jnp.int32),
        pltpu.VMEM((_CHUNK,), jnp.float32),
        pltpu.SemaphoreType.DMA,
        pltpu.SemaphoreType.DMA,
    ],
)
def _atomref_sc(types_hbm, table_hbm, out_hbm, table_v, idx_v, out_v,
                sem_t, sem_i):
    wid = lax.axis_index("s") * _NC + lax.axis_index("c")
    base = lax.min(wid * _CHUNK, _N - _CHUNK)

    ct = pltpu.async_copy(table_hbm, table_v, sem_t)
    ci = pltpu.async_copy(types_hbm.at[pl.ds(base, _CHUNK)], idx_v, sem_i)
    ct.wait()
    ci.wait()

    @plsc.parallel_loop(0, _CHUNK, _L, unroll=8)
    def _body(i):
        idx = idx_v[pl.ds(i, _L)]
        out_v[pl.ds(i, _L)] = plsc.load_gather(table_v, [idx])

    pltpu.sync_copy(out_v, out_hbm.at[pl.ds(base, _CHUNK)])


def kernel(atom_types, elemental_energies):
    return _atomref_sc(atom_types.astype(jnp.int32), elemental_energies)
